# baseline (speedup 1.0000x reference)
import functools

import jax
import jax.numpy as jnp
from jax import lax
from jax.experimental import pallas as pl
from jax.experimental.pallas import tpu as pltpu
from jax.experimental.pallas import tpu_sc as plsc

NTOK = 16384
NTAGS = 16
NSUB = 16
CHUNK = 128
PER_TILE = NTOK // NSUB
NCHUNK = PER_TILE // CHUNK


def _bow_body(x_hbm, table_hbm, bias_hbm, out_hbm,
              idx_v, rows_v, part_v, allp_v, bias_v, out_v, shared, sem):
  sid = lax.axis_index("s")

  pltpu.sync_copy(x_hbm.at[pl.ds(sid * NCHUNK, NCHUNK)], idx_v)

  descs = [
      pltpu.async_copy(
          table_hbm.at[idx_v.at[j]],
          rows_v.at[pl.ds(j * CHUNK, CHUNK)],
          sem,
      )
      for j in range(NCHUNK)
  ]
  for d in descs:
    d.wait()

  zero = jnp.zeros((NTAGS,), jnp.float32)

  def acc_fn(i, accs):
    a0, a1, a2, a3 = accs
    b = i * 4
    return (a0 + rows_v[b, :], a1 + rows_v[b + 1, :],
            a2 + rows_v[b + 2, :], a3 + rows_v[b + 3, :])

  a0, a1, a2, a3 = lax.fori_loop(0, PER_TILE // 4, acc_fn,
                                 (zero, zero, zero, zero))
  part_v[0, :] = (a0 + a1) + (a2 + a3)

  pltpu.sync_copy(part_v, shared.at[pl.ds(sid, 1)])
  plsc.subcore_barrier()

  @pl.when(sid == 0)
  def _():
    pltpu.sync_copy(shared, allp_v)
    pltpu.sync_copy(bias_hbm, bias_v)
    tot = bias_v[...]
    for r in range(NSUB):
      tot = tot + allp_v[r, :]
    out_v[0, :] = tot
    pltpu.sync_copy(out_v, out_hbm)


@functools.partial(jax.jit)
def _bow(x2, table, bias):
  mesh = plsc.VectorSubcoreMesh(
      core_axis_name="c", subcore_axis_name="s", num_cores=1)
  call = functools.partial(
      pl.kernel,
      out_type=jax.ShapeDtypeStruct((1, NTAGS), jnp.float32),
      mesh=mesh,
      compiler_params=pltpu.CompilerParams(use_tc_tiling_on_sc=False),
      scratch_types=[
          pltpu.VMEM((NCHUNK, CHUNK), jnp.int32),
          pltpu.VMEM((PER_TILE, NTAGS), jnp.float32),
          pltpu.VMEM((1, NTAGS), jnp.float32),
          pltpu.VMEM((NSUB, NTAGS), jnp.float32),
          pltpu.VMEM((NTAGS,), jnp.float32),
          pltpu.VMEM((1, NTAGS), jnp.float32),
          pltpu.VMEM_SHARED((NSUB, NTAGS), jnp.float32),
          pltpu.SemaphoreType.DMA,
      ],
  )
  return call(_bow_body)(x2, table, bias)


def kernel(x, table, bias):
  x2 = x.astype(jnp.int32).reshape(NTOK // CHUNK, CHUNK)
  return _bow(x2, table, bias)


# speedup vs baseline: 1.1452x; 1.1452x over previous
import functools

import jax
import jax.numpy as jnp
from jax import lax
from jax.experimental import pallas as pl
from jax.experimental.pallas import tpu as pltpu
from jax.experimental.pallas import tpu_sc as plsc

NTOK = 16384
NTAGS = 16
NSUB = 16
PER_TILE = NTOK // NSUB
NWORDS = 1000000
COL = 128
NCOLS = NWORDS // COL
TAIL = NWORDS - NCOLS * COL
NBUF = 4
ZBLK = 8192
CPAD = NCOLS * COL + COL


def _bow_body(x_hbm, tab_hbm, tail_hbm, bias_hbm, out_hbm,
              idx_v, ones_v, zero_v, buf_v, cnt_v, amat_v, allm_v, m_v, mt_v,
              bias_v, out_v, counts_sp, shared, shared_m, sems, csems,
              sem, zsem):
  sid = lax.axis_index("s")
  zvec = jnp.zeros((NTAGS,), jnp.float32)

  def zstore(i, c):
    zero_v[pl.ds(i * NTAGS, NTAGS)] = zvec
    return c
  lax.fori_loop(0, ZBLK // NTAGS, zstore, 0)
  for g in range(COL // NTAGS):
    ones_v[pl.ds(g * NTAGS, NTAGS)] = jnp.ones((NTAGS,), jnp.float32)
  zshare = 62464
  def zcopy(i, c):
    pltpu.sync_copy(zero_v.at[pl.ds(0, ZBLK)],
                    counts_sp.at[pl.ds(sid * zshare + i * ZBLK, ZBLK)])
    return c
  lax.fori_loop(0, zshare // ZBLK, zcopy, 0)
  pltpu.sync_copy(zero_v.at[pl.ds(0, zshare - (zshare // ZBLK) * ZBLK)],
                  counts_sp.at[pl.ds(sid * zshare + (zshare // ZBLK) * ZBLK,
                                     zshare - (zshare // ZBLK) * ZBLK)])
  @pl.when(sid == 0)
  def _():
    rest = CPAD - NSUB * zshare
    pltpu.sync_copy(zero_v.at[pl.ds(0, rest)],
                    counts_sp.at[pl.ds(NSUB * zshare, rest)])
  plsc.subcore_barrier()

  pltpu.sync_copy(
      x_hbm.at[pl.ds(sid * (PER_TILE // COL), PER_TILE // COL)], idx_v)
  for k in range(PER_TILE // COL):
    pltpu.sync_copy(ones_v, counts_sp.at[idx_v.at[k]], add=True)
  plsc.subcore_barrier()

  start = sid * (NCOLS // NSUB)
  ncols = NCOLS // NSUB

  def fire(k):
    j = start + lax.rem(k, ncols)
    off = j * COL
    b = lax.rem(k, NBUF)
    pltpu.async_copy(tab_hbm.at[:, pl.ds(off, COL)], buf_v.at[b], sems.at[b])
    pltpu.async_copy(counts_sp.at[pl.ds(off, COL)], cnt_v.at[b], csems.at[b])

  def wait_one(k):
    b = lax.rem(k, NBUF)
    pltpu.make_async_copy(tab_hbm.at[:, pl.ds(0, COL)],
                          buf_v.at[b], sems.at[b]).wait()
    pltpu.make_async_copy(counts_sp.at[pl.ds(0, COL)], cnt_v.at[b],
                          csems.at[b]).wait()

  def col_fn(k, accs):
    fire(k)
    wait_one(k)
    b = lax.rem(k, NBUF)
    accs = list(accs)
    cvecs = [cnt_v[b, pl.ds(g * NTAGS, NTAGS)] for g in range(COL // NTAGS)]
    for p in range(NTAGS):
      acc = accs[p]
      for g in range(COL // NTAGS):
        acc = acc + buf_v[b, p, pl.ds(g * NTAGS, NTAGS)] * cvecs[g]
      accs[p] = acc
    return tuple(accs)

  accs = lax.fori_loop(0, ncols, col_fn, (zvec,) * NTAGS)

  for p in range(NTAGS):
    amat_v[pl.ds(p * NTAGS, NTAGS)] = accs[p]

  @pl.when(sid == 0)
  def _():
    for j in range(NSUB * (NCOLS // NSUB), NCOLS):
      d1 = pltpu.async_copy(tab_hbm.at[:, pl.ds(j * COL, COL)],
                            buf_v.at[0], sem)
      d2 = pltpu.async_copy(counts_sp.at[pl.ds(j * COL, COL)],
                            cnt_v.at[0], zsem)
      d1.wait()
      d2.wait()
      for p in range(NTAGS):
        acc = amat_v[pl.ds(p * NTAGS, NTAGS)]
        for g in range(COL // NTAGS):
          acc = acc + buf_v[0, p, pl.ds(g * NTAGS, NTAGS)] * \
              cnt_v[0, pl.ds(g * NTAGS, NTAGS)]
        amat_v[pl.ds(p * NTAGS, NTAGS)] = acc
    d1 = pltpu.async_copy(tail_hbm, buf_v.at[0], sem)
    d2 = pltpu.async_copy(counts_sp.at[pl.ds(NCOLS * COL, COL)],
                          cnt_v.at[0], zsem)
    d1.wait()
    d2.wait()
    for p in range(NTAGS):
      acc = amat_v[pl.ds(p * NTAGS, NTAGS)]
      for g in range(COL // NTAGS):
        acc = acc + buf_v[0, p, pl.ds(g * NTAGS, NTAGS)] * \
            cnt_v[0, pl.ds(g * NTAGS, NTAGS)]
      amat_v[pl.ds(p * NTAGS, NTAGS)] = acc

  pltpu.sync_copy(amat_v, shared.at[pl.ds(sid * NTAGS * NTAGS, NTAGS * NTAGS)])
  plsc.subcore_barrier()

  @pl.when(sid == 0)
  def _():
    pltpu.sync_copy(shared, allm_v)
    for p in range(NTAGS):
      acc = allm_v[pl.ds(p * NTAGS, NTAGS)]
      for t in range(1, NSUB):
        acc = acc + allm_v[pl.ds((t * NTAGS + p) * NTAGS, NTAGS)]
      m_v[p, :] = acc
    pltpu.sync_copy(m_v, out_hbm)


@functools.partial(jax.jit)
def _bow(x2, tab_t, tail128, bias):
  mesh = plsc.VectorSubcoreMesh(
      core_axis_name="c", subcore_axis_name="s", num_cores=1)
  call = functools.partial(
      pl.kernel,
      out_type=jax.ShapeDtypeStruct((NTAGS, NTAGS), jnp.float32),
      mesh=mesh,
      scratch_types=[
          pltpu.VMEM((PER_TILE // COL, COL), jnp.int32),
          pltpu.VMEM((COL,), jnp.float32),
          pltpu.VMEM((ZBLK,), jnp.float32),
          pltpu.VMEM((NBUF, NTAGS, COL), jnp.float32),
          pltpu.VMEM((NBUF, COL), jnp.float32),
          pltpu.VMEM((NTAGS * NTAGS,), jnp.float32),
          pltpu.VMEM((NSUB * NTAGS * NTAGS,), jnp.float32),
          pltpu.VMEM((NTAGS, NTAGS), jnp.float32),
          pltpu.VMEM((NTAGS, NTAGS), jnp.float32),
          pltpu.VMEM((NTAGS,), jnp.float32),
          pltpu.VMEM((1, NTAGS), jnp.float32),
          pltpu.VMEM_SHARED((CPAD,), jnp.float32),
          pltpu.VMEM_SHARED((NSUB * NTAGS * NTAGS,), jnp.float32),
          pltpu.VMEM_SHARED((NTAGS, NTAGS), jnp.float32),
          pltpu.SemaphoreType.DMA((NBUF,)),
          pltpu.SemaphoreType.DMA((NBUF,)),
          pltpu.SemaphoreType.DMA,
          pltpu.SemaphoreType.DMA,
      ],
  )
  return call(_bow_body)(x2, tab_t, tail128, bias)


def kernel(x, table, bias):
  x2 = x.astype(jnp.int32).reshape(NTOK // COL, COL)
  tab_t = jnp.swapaxes(table, 0, 1)
  tail128 = jnp.swapaxes(jnp.concatenate(
      [table[NCOLS * COL:], jnp.zeros((COL - TAIL, NTAGS), table.dtype)],
      axis=0), 0, 1)
  m = _bow(x2, tab_t, tail128, bias)
  return (m.sum(axis=1) + bias).reshape(1, NTAGS)
